# baseline (speedup 1.0000x reference)
import functools

import jax
import jax.numpy as jnp
from jax import lax
from jax.experimental import pallas as pl
from jax.experimental.pallas import tpu as pltpu
from jax.experimental.pallas import tpu_sc as plsc

BATCH = 16384
EMB = 64
NC = 2
NS = 16
L = 16
NW = NC * NS
BPW = BATCH // NW
CHUNK = 128
NCHUNK = BPW // CHUNK
NG = BPW // L


def _body(user_hbm, ad_hbm, utab_hbm, atab_hbm, fcw_hbm, fcb_hbm, out_hbm,
          uidx_v, aidx_v, urows_v, arows_v, fcw_v, fcb_v, outbuf_v, sem):
    wid = lax.axis_index("s") * NC + lax.axis_index("c")
    crow = wid * NCHUNK

    pltpu.sync_copy(user_hbm.at[pl.ds(crow, NCHUNK)], uidx_v)
    pltpu.sync_copy(ad_hbm.at[pl.ds(crow, NCHUNK)], aidx_v)
    pltpu.sync_copy(fcw_hbm, fcw_v)
    pltpu.sync_copy(fcb_hbm, fcb_v)

    handles = []
    for j in range(NCHUNK):
        handles.append(pltpu.async_copy(
            utab_hbm.at[uidx_v.at[j]],
            urows_v.at[pl.ds(j * CHUNK, CHUNK)], sem))
        handles.append(pltpu.async_copy(
            atab_hbm.at[aidx_v.at[j]],
            arows_v.at[pl.ds(j * CHUNK, CHUNK)], sem))
    for h in handles:
        h.wait()

    iot = lax.iota(jnp.int32, L)
    wv = fcw_v[...]
    bv = fcb_v[...]

    def group(g, carry):
        row16 = g * L + iot
        acc_ua = jnp.zeros((L,), jnp.float32)
        acc_uu = jnp.zeros((L,), jnp.float32)
        acc_aa = jnp.zeros((L,), jnp.float32)
        for d in range(EMB):
            dsp = jnp.full((L,), d, jnp.int32)
            u = plsc.load_gather(urows_v, [row16, dsp])
            a = plsc.load_gather(arows_v, [row16, dsp])
            acc_ua = acc_ua + u * a
            acc_uu = acc_uu + u * u
            acc_aa = acc_aa + a * a
        x = jnp.maximum(acc_uu * acc_aa, jnp.float32(1e-30))
        i = lax.bitcast_convert_type(x, jnp.int32)
        i = jnp.int32(0x5F3759DF) - lax.shift_right_logical(i, 1)
        y = lax.bitcast_convert_type(i, jnp.float32)
        for _ in range(3):
            y = y * (jnp.float32(1.5) - jnp.float32(0.5) * x * y * y)
        dot = acc_ua * y
        z = dot * wv + bv
        s = jnp.float32(1.0) / (jnp.float32(1.0) + jnp.exp(-z))
        outbuf_v[pl.ds(g * L, L)] = s
        return carry

    lax.fori_loop(0, NG, group, 0)

    pltpu.sync_copy(outbuf_v, out_hbm.at[pl.ds(wid * BPW, BPW)])


@jax.jit
def _run(user2, ad2, user_table, ad_table, wvec, bvec):
    mesh = plsc.VectorSubcoreMesh(core_axis_name="c", subcore_axis_name="s")
    k = functools.partial(
        pl.kernel,
        mesh=mesh,
        compiler_params=pltpu.CompilerParams(
            use_tc_tiling_on_sc=False, needs_layout_passes=False),
        out_type=jax.ShapeDtypeStruct((BATCH,), jnp.float32),
        scratch_types=[
            pltpu.VMEM((NCHUNK, CHUNK), jnp.int32),
            pltpu.VMEM((NCHUNK, CHUNK), jnp.int32),
            pltpu.VMEM((BPW, EMB), jnp.float32),
            pltpu.VMEM((BPW, EMB), jnp.float32),
            pltpu.VMEM((L,), jnp.float32),
            pltpu.VMEM((L,), jnp.float32),
            pltpu.VMEM((BPW,), jnp.float32),
            pltpu.SemaphoreType.DMA,
        ],
    )(_body)
    return k(user2, ad2, user_table, ad_table, wvec, bvec)


def kernel(user, ad, user_table, ad_table, fc_w, fc_b):
    user2 = user.astype(jnp.int32).reshape(NW * NCHUNK, CHUNK)
    ad2 = ad.astype(jnp.int32).reshape(NW * NCHUNK, CHUNK)
    wvec = jnp.broadcast_to(fc_w.reshape(()), (L,)).astype(jnp.float32)
    bvec = jnp.broadcast_to(fc_b.reshape(()), (L,)).astype(jnp.float32)
    out = _run(user2, ad2, user_table, ad_table, wvec, bvec)
    return out.reshape(BATCH, 1)


# speedup vs baseline: 1.5199x; 1.5199x over previous
import functools

import jax
import jax.numpy as jnp
from jax import lax
from jax.experimental import pallas as pl
from jax.experimental.pallas import tpu as pltpu
from jax.experimental.pallas import tpu_sc as plsc

BATCH = 16384
EMB = 64
NC = 2
NS = 16
L = 16
NW = NC * NS
BPW = BATCH // NW
CHR = 256
NPASS = BPW // CHR
NGC = CHR // L


def _body(user_hbm, ad_hbm, utab_hbm, atab_hbm, fcw_hbm, fcb_hbm, out_hbm,
          uidx_v, aidx_v, urows_v, arows_v, fcw_v, fcb_v, outbuf_v, sem):
    wid = lax.axis_index("s") * NC + lax.axis_index("c")
    base = wid * BPW

    pltpu.sync_copy(user_hbm.at[pl.ds(base, BPW)], uidx_v)
    pltpu.sync_copy(ad_hbm.at[pl.ds(base, BPW)], aidx_v)
    pltpu.sync_copy(fcw_hbm, fcw_v)
    pltpu.sync_copy(fcb_hbm, fcb_v)

    iot = lax.iota(jnp.int32, L)
    wv = fcw_v[...]
    bv = fcb_v[...]

    for p in range(NPASS):
        def fetch(g, carry, p=p):
            uidx = uidx_v[pl.ds((p * NGC + g) * L, L)]
            aidx = aidx_v[pl.ds((p * NGC + g) * L, L)]
            handles = []
            for l in range(L):
                k = g * L + l
                handles.append(pltpu.make_async_copy(
                    utab_hbm.at[uidx[l]], urows_v.at[k], sem))
                handles.append(pltpu.make_async_copy(
                    atab_hbm.at[aidx[l]], arows_v.at[k], sem))
            for h in handles:
                h.start()
            for h in handles:
                h.wait()
            return carry

        lax.fori_loop(0, NGC, fetch, 0)

        def group(g, carry, p=p):
            row16 = g * L + iot
            acc_ua = jnp.zeros((L,), jnp.float32)
            acc_uu = jnp.zeros((L,), jnp.float32)
            acc_aa = jnp.zeros((L,), jnp.float32)
            for d in range(EMB):
                dsp = jnp.full((L,), d, jnp.int32)
                u = plsc.load_gather(urows_v, [row16, dsp])
                a = plsc.load_gather(arows_v, [row16, dsp])
                acc_ua = acc_ua + u * a
                acc_uu = acc_uu + u * u
                acc_aa = acc_aa + a * a
            x = jnp.maximum(acc_uu * acc_aa, jnp.float32(1e-30))
            i = lax.bitcast_convert_type(x, jnp.int32)
            i = jnp.int32(0x5F3759DF) - lax.shift_right_logical(i, 1)
            y = lax.bitcast_convert_type(i, jnp.float32)
            for _ in range(3):
                y = y * (jnp.float32(1.5) - jnp.float32(0.5) * x * y * y)
            dot = acc_ua * y
            z = dot * wv + bv
            s = jnp.float32(1.0) / (jnp.float32(1.0) + jnp.exp(-z))
            outbuf_v[pl.ds((p * NGC + g) * L, L)] = s
            return carry

        lax.fori_loop(0, NGC, group, 0)

    pltpu.sync_copy(outbuf_v, out_hbm.at[pl.ds(base, BPW)])


@jax.jit
def _run(user, ad, user_table, ad_table, wvec, bvec):
    mesh = plsc.VectorSubcoreMesh(core_axis_name="c", subcore_axis_name="s")
    k = functools.partial(
        pl.kernel,
        mesh=mesh,
        compiler_params=pltpu.CompilerParams(needs_layout_passes=False),
        out_type=jax.ShapeDtypeStruct((BATCH,), jnp.float32),
        scratch_types=[
            pltpu.VMEM((BPW,), jnp.int32),
            pltpu.VMEM((BPW,), jnp.int32),
            pltpu.VMEM((CHR, EMB), jnp.float32),
            pltpu.VMEM((CHR, EMB), jnp.float32),
            pltpu.VMEM((L,), jnp.float32),
            pltpu.VMEM((L,), jnp.float32),
            pltpu.VMEM((BPW,), jnp.float32),
            pltpu.SemaphoreType.DMA,
        ],
    )(_body)
    return k(user, ad, user_table, ad_table, wvec, bvec)


def kernel(user, ad, user_table, ad_table, fc_w, fc_b):
    user = user.astype(jnp.int32)
    ad = ad.astype(jnp.int32)
    wvec = jnp.broadcast_to(fc_w.reshape(()), (L,)).astype(jnp.float32)
    bvec = jnp.broadcast_to(fc_b.reshape(()), (L,)).astype(jnp.float32)
    out = _run(user, ad, user_table, ad_table, wvec, bvec)
    return out.reshape(BATCH, 1)
